# baseline (speedup 1.0000x reference)
import jax
import jax.numpy as jnp
from jax.experimental import pallas as pl
from jax.experimental.pallas import tpu as pltpu


def _make_kernel(bm, half_rows):
    def _k(a_ref, b_ref, x_ref, w_ref, ws_ref, o_ref, s_ref):
        i = pl.program_id(0)

        @pl.when(i == 0)
        def _():
            s_ref[...] = jnp.dot(
                x_ref[...], w_ref[...], preferred_element_type=jnp.float32
            )

        ws = ws_ref[...]
        row_a = i * bm
        row_b = half_rows + i * bm
        acc_a = jnp.dot(a_ref[...], s_ref[...], preferred_element_type=jnp.float32)
        self_a = s_ref[pl.ds(row_a, bm), :] + jnp.dot(
            x_ref[pl.ds(row_a, bm), :], ws, preferred_element_type=jnp.float32
        )
        o_ref[0] = jnp.maximum(acc_a + self_a, 0.0)
        acc_b = jnp.dot(b_ref[...], s_ref[...], preferred_element_type=jnp.float32)
        self_b = s_ref[pl.ds(row_b, bm), :] + jnp.dot(
            x_ref[pl.ds(row_b, bm), :], ws, preferred_element_type=jnp.float32
        )
        o_ref[1] = jnp.maximum(acc_b + self_b, 0.0)

    return _k


def _pick_tile(n, candidates):
    for c in candidates:
        if n % c == 0:
            return c
    return n


@jax.jit
def kernel(x, adj, W, W_self):
    N, din = x.shape
    dout = W.shape[1]
    bm = _pick_tile(N // 2, (200, 100, 50, 8))
    half_steps = N // (2 * bm)

    out3 = pl.pallas_call(
        _make_kernel(bm, N // 2),
        grid=(half_steps,),
        in_specs=[
            pl.BlockSpec((bm, N), lambda i: (i, 0)),
            pl.BlockSpec((bm, N), lambda i, _h=half_steps: (i + _h, 0)),
            pl.BlockSpec((N, din), lambda i: (0, 0)),
            pl.BlockSpec((din, dout), lambda i: (0, 0)),
            pl.BlockSpec((din, dout), lambda i: (0, 0)),
        ],
        out_specs=pl.BlockSpec((2, bm, dout), lambda i: (0, i, 0)),
        out_shape=jax.ShapeDtypeStruct((2, N // 2, dout), jnp.float32),
        scratch_shapes=[pltpu.VMEM((N, dout), jnp.float32)],
        compiler_params=pltpu.CompilerParams(
            dimension_semantics=("arbitrary",),
        ),
    )(adj, adj, x, W, W_self)
    return out3.reshape(N, dout)
